# baseline (speedup 1.0000x reference)
import functools

import jax
import jax.numpy as jnp
import numpy as np
from jax import lax
from jax.experimental import pallas as pl
from jax.experimental.pallas import tpu as pltpu
from jax.experimental.pallas import tpu_sc as plsc

_INFO = plsc.get_sparse_core_info()
_NC = _INFO.num_cores
_NS = _INFO.num_subcores
_NW = _NC * _NS

_PB = 2
_LANES = 128
_CG = 4


def _gather_sc(idx_flat, table, n_pairs, d):
    dt = table.shape[1]
    pairs_per_w = n_pairs // _NW
    n_stages = pairs_per_w // _PB
    rows_per_stage = _PB * _LANES
    obuf_len = _CG * _PB * 8 * _LANES
    mesh = plsc.VectorSubcoreMesh(core_axis_name="c", subcore_axis_name="s")

    @functools.partial(
        pl.kernel,
        mesh=mesh,
        out_type=jax.ShapeDtypeStruct((n_pairs // _LANES, _CG, _LANES * 8 * _LANES),
                                      jnp.float32),
        scratch_types=(
            [pltpu.VMEM((rows_per_stage,), jnp.int32) for _ in range(2)]
            + [pltpu.VMEM((rows_per_stage, dt), jnp.float32) for _ in range(2)]
            + [pltpu.VMEM((obuf_len + _LANES + 16,), jnp.float32)
               for _ in range(2)]
            + [pltpu.SemaphoreType.DMA for _ in range(6)]
        ),
        compiler_params=pltpu.CompilerParams(use_tc_tiling_on_sc=False,
                                             needs_layout_passes=False),
    )
    def k(idx_hbm, table_hbm, out_hbm, *scr):
        idx_bufs, row_bufs, obufs = scr[0:2], scr[2:4], scr[4:6]
        isems, gsems, osems = scr[6:8], scr[8:10], scr[10:12]

        wid = lax.axis_index("s") * _NC + lax.axis_index("c")
        p0 = wid * pairs_per_w
        lane = lax.iota(jnp.int32, 16)

        def idx_start(s, b):
            pltpu.async_copy(
                idx_hbm.at[pl.ds((p0 + s * _PB) * _LANES, rows_per_stage)],
                idx_bufs[b], isems[b])

        def idx_wait(b):
            pltpu.make_async_copy(
                idx_hbm.at[pl.ds(0, rows_per_stage)], idx_bufs[b],
                isems[b]).wait()

        def gather_start(b):
            pltpu.async_copy(table_hbm.at[idx_bufs[b]], row_bufs[b], gsems[b])

        def gather_wait(b):
            pltpu.make_async_copy(
                table_hbm.at[idx_bufs[b]], row_bufs[b], gsems[b]).wait()

        def out_start(s, b):
            p = p0 + s * _PB
            h = p // _LANES
            btile = p % _LANES
            for c4 in range(_CG):
                pltpu.async_copy(
                    obufs[b].at[pl.ds(c4 * _PB * 8 * _LANES, _PB * 8 * _LANES)],
                    out_hbm.at[h, c4, pl.ds(btile * 8 * _LANES,
                                            _PB * 8 * _LANES)],
                    osems[b])

        def out_wait(b):
            for c4 in range(_CG):
                pltpu.make_async_copy(
                    obufs[b].at[pl.ds(0, _PB * 8 * _LANES)],
                    out_hbm.at[0, 0, pl.ds(0, _PB * 8 * _LANES)],
                    osems[b]).wait()

        iv_cb = []
        for ch in range(d // 16):
            row = []
            for blo in range(8):
                cc = lane + ch * 16
                row.append((cc >> 3) * (_PB * 8 * _LANES)
                           + (cc & 7) * _LANES + blo)
            iv_cb.append(row)
        window = (d // 8 - 1) * (_PB * 8 * _LANES) + 7 * _LANES + 8 + 16

        def transpose(b):
            def lbody(bl8, carry):
                base = bl8 * 8
                for j2 in range(_PB):
                    off = j2 * (8 * _LANES) + base
                    dst = obufs[b].at[pl.ds(off, window)]
                    for ch in range(d // 16):
                        vecs = [row_bufs[b][j2 * _LANES + base + blo,
                                            pl.ds(ch * 16, 16)]
                                for blo in range(8)]
                        for blo in range(8):
                            plsc.store_scatter(dst, [iv_cb[ch][blo]],
                                               vecs[blo])
                return carry
            lax.fori_loop(0, _LANES // 8, lbody, 0)

        idx_start(0, 0)
        idx_start(1, 1)
        idx_wait(0)
        gather_start(0)

        def outer_body(o, carry):
            for b in range(2):
                s = o * 2 + b
                nb = 1 - b

                @pl.when(s + 1 < n_stages)
                def _():
                    idx_wait(nb)
                    gather_start(nb)

                gather_wait(b)

                @pl.when(s >= 2)
                def _():
                    out_wait(b)

                transpose(b)
                out_start(s, b)

                @pl.when(s + 2 < n_stages)
                def _():
                    idx_start(s + 2, b)
            return carry

        lax.fori_loop(0, n_stages // 2, outer_body, 0)
        out_wait(0)
        out_wait(1)

    return k(idx_flat, table)


def kernel(tokens, embeddings):
    bsz, hist = tokens.shape
    v, d = embeddings.shape
    n_pairs = hist * (bsz // _LANES)
    idx_flat = jnp.transpose(tokens).reshape(bsz * hist).astype(jnp.int32)
    embeddings = jnp.pad(embeddings, ((0, 0), (0, 128 - d)))
    out_lin = _gather_sc(idx_flat, embeddings, n_pairs, d)
    out = out_lin.reshape(hist, _CG, bsz // _LANES, 8, _LANES)
    out = out.transpose(2, 4, 0, 1, 3).reshape(bsz, hist, d)
    return out


# speedup vs baseline: 1.0260x; 1.0260x over previous
import functools

import jax
import jax.numpy as jnp
import numpy as np
from jax import lax
from jax.experimental import pallas as pl
from jax.experimental.pallas import tpu as pltpu
from jax.experimental.pallas import tpu_sc as plsc

_INFO = plsc.get_sparse_core_info()
_NC = _INFO.num_cores
_NS = _INFO.num_subcores
_NW = _NC * _NS

_PB = 5
_LANES = 128
_CG = 4


def _gather_sc(idx_flat, table, n_pairs, d):
    pairs_per_w = n_pairs // _NW
    n_stages = pairs_per_w // _PB
    rows_per_stage = _PB * _LANES
    obuf_len = _CG * _PB * 8 * _LANES
    mesh = plsc.VectorSubcoreMesh(core_axis_name="c", subcore_axis_name="s")

    @functools.partial(
        pl.kernel,
        mesh=mesh,
        out_type=jax.ShapeDtypeStruct((n_pairs // _LANES, _CG, _LANES * 8 * _LANES),
                                      jnp.float32),
        scratch_types=(
            [pltpu.VMEM((rows_per_stage,), jnp.int32) for _ in range(2)]
            + [pltpu.VMEM((rows_per_stage, d), jnp.float32) for _ in range(2)]
            + [pltpu.VMEM((obuf_len + _LANES + 16,), jnp.float32)
               for _ in range(2)]
            + [pltpu.SemaphoreType.DMA for _ in range(6)]
        ),
        compiler_params=pltpu.CompilerParams(use_tc_tiling_on_sc=False,
                                             needs_layout_passes=False),
    )
    def k(idx_hbm, table_hbm, out_hbm, *scr):
        idx_bufs, row_bufs, obufs = scr[0:2], scr[2:4], scr[4:6]
        isems, gsems, osems = scr[6:8], scr[8:10], scr[10:12]

        wid = lax.axis_index("s") * _NC + lax.axis_index("c")
        p0 = wid * pairs_per_w
        lane = lax.iota(jnp.int32, 16)

        def idx_start(s, b):
            pltpu.async_copy(
                idx_hbm.at[pl.ds((p0 + s * _PB) * _LANES, rows_per_stage)],
                idx_bufs[b], isems[b])

        def idx_wait(b):
            pltpu.make_async_copy(
                idx_hbm.at[pl.ds(0, rows_per_stage)], idx_bufs[b],
                isems[b]).wait()

        def gather_start(b):
            pltpu.async_copy(table_hbm.at[idx_bufs[b]], row_bufs[b], gsems[b])

        def gather_wait(b):
            pltpu.make_async_copy(
                table_hbm.at[idx_bufs[b]], row_bufs[b], gsems[b]).wait()

        def out_start(s, b):
            p = p0 + s * _PB
            h = p // _LANES
            btile = p % _LANES
            for c4 in range(_CG):
                pltpu.async_copy(
                    obufs[b].at[pl.ds(c4 * _PB * 8 * _LANES, _PB * 8 * _LANES)],
                    out_hbm.at[h, c4, pl.ds(btile * 8 * _LANES,
                                            _PB * 8 * _LANES)],
                    osems[b])

        def out_wait(b):
            for c4 in range(_CG):
                pltpu.make_async_copy(
                    obufs[b].at[pl.ds(0, _PB * 8 * _LANES)],
                    out_hbm.at[0, 0, pl.ds(0, _PB * 8 * _LANES)],
                    osems[b]).wait()

        iv_cb = []
        for ch in range(d // 16):
            row = []
            for blo in range(8):
                cc = lane + ch * 16
                row.append((cc >> 3) * (_PB * 8 * _LANES)
                           + (cc & 7) * _LANES + blo)
            iv_cb.append(row)
        window = (d // 8 - 1) * (_PB * 8 * _LANES) + 7 * _LANES + 8 + 16

        def transpose(b):
            def lbody(bl8, carry):
                base = bl8 * 8
                for j2 in range(_PB):
                    off = j2 * (8 * _LANES) + base
                    dst = obufs[b].at[pl.ds(off, window)]
                    vecs = [[row_bufs[b][j2 * _LANES + base + blo,
                                         pl.ds(ch * 16, 16)]
                             for blo in range(8)] for ch in range(d // 16)]
                    for ch in range(d // 16):
                        for blo in range(8):
                            plsc.store_scatter(dst, [iv_cb[ch][blo]],
                                               vecs[ch][blo])
                return carry
            lax.fori_loop(0, _LANES // 8, lbody, 0)

        idx_start(0, 0)
        idx_start(1, 1)
        idx_wait(0)
        gather_start(0)

        def outer_body(o, carry):
            for b in range(2):
                s = o * 2 + b
                nb = 1 - b

                @pl.when(s + 1 < n_stages)
                def _():
                    idx_wait(nb)
                    gather_start(nb)

                gather_wait(b)

                @pl.when(s >= 2)
                def _():
                    out_wait(b)

                transpose(b)
                out_start(s, b)

                @pl.when(s + 2 < n_stages)
                def _():
                    idx_start(s + 2, b)
            return carry

        lax.fori_loop(0, n_stages // 2, outer_body, 0)
        out_wait(0)
        out_wait(1)

    return k(idx_flat, table)


def kernel(tokens, embeddings):
    bsz, hist = tokens.shape
    v, d = embeddings.shape
    n_pairs = hist * (bsz // _LANES)
    idx_flat = jnp.transpose(tokens).reshape(bsz * hist).astype(jnp.int32)
    out_lin = _gather_sc(idx_flat, embeddings, n_pairs, d)
    out = out_lin.reshape(hist, _CG, bsz // _LANES, 8, _LANES)
    out = out.transpose(2, 4, 0, 1, 3).reshape(bsz, hist, d)
    return out


# speedup vs baseline: 1.0472x; 1.0207x over previous
import functools

import jax
import jax.numpy as jnp
import numpy as np
from jax import lax
from jax.experimental import pallas as pl
from jax.experimental.pallas import tpu as pltpu
from jax.experimental.pallas import tpu_sc as plsc

_INFO = plsc.get_sparse_core_info()
_NC = _INFO.num_cores
_NS = _INFO.num_subcores
_NW = _NC * _NS

_PB = 4
_LANES = 128
_CG = 4


def _gather_sc(idx_flat, table, n_pairs, d):
    pairs_per_w = n_pairs // _NW
    n_stages = pairs_per_w // _PB
    rows_per_stage = _PB * _LANES
    obuf_len = _CG * _PB * 8 * _LANES
    mesh = plsc.VectorSubcoreMesh(core_axis_name="c", subcore_axis_name="s")

    @functools.partial(
        pl.kernel,
        mesh=mesh,
        out_type=jax.ShapeDtypeStruct((n_pairs // _LANES, _CG, _LANES * 8 * _LANES),
                                      jnp.float32),
        scratch_types=(
            [pltpu.VMEM((rows_per_stage,), jnp.int32) for _ in range(2)]
            + [pltpu.VMEM((rows_per_stage, d), jnp.float32) for _ in range(2)]
            + [pltpu.VMEM((obuf_len + _LANES + 16,), jnp.float32)
               for _ in range(2)]
            + [pltpu.SemaphoreType.DMA for _ in range(6)]
        ),
        compiler_params=pltpu.CompilerParams(use_tc_tiling_on_sc=False,
                                             needs_layout_passes=False),
    )
    def k(idx_hbm, table_hbm, out_hbm, *scr):
        idx_bufs, row_bufs, obufs = scr[0:2], scr[2:4], scr[4:6]
        isems, gsems, osems = scr[6:8], scr[8:10], scr[10:12]

        wid = lax.axis_index("s") * _NC + lax.axis_index("c")
        p0 = wid * pairs_per_w
        lane = lax.iota(jnp.int32, 16)

        def idx_start(s, b):
            pltpu.async_copy(
                idx_hbm.at[pl.ds((p0 + s * _PB) * _LANES, rows_per_stage)],
                idx_bufs[b], isems[b])

        def idx_wait(b):
            pltpu.make_async_copy(
                idx_hbm.at[pl.ds(0, rows_per_stage)], idx_bufs[b],
                isems[b]).wait()

        def gather_start(b):
            pltpu.async_copy(table_hbm.at[idx_bufs[b]], row_bufs[b], gsems[b])

        def gather_wait(b):
            pltpu.make_async_copy(
                table_hbm.at[idx_bufs[b]], row_bufs[b], gsems[b]).wait()

        def out_start(s, b):
            p = p0 + s * _PB
            h = p // _LANES
            btile = p % _LANES
            for c4 in range(_CG):
                pltpu.async_copy(
                    obufs[b].at[pl.ds(c4 * _PB * 8 * _LANES, _PB * 8 * _LANES)],
                    out_hbm.at[h, c4, pl.ds(btile * 8 * _LANES,
                                            _PB * 8 * _LANES)],
                    osems[b])

        def out_wait(b):
            for c4 in range(_CG):
                pltpu.make_async_copy(
                    obufs[b].at[pl.ds(0, _PB * 8 * _LANES)],
                    out_hbm.at[0, 0, pl.ds(0, _PB * 8 * _LANES)],
                    osems[b]).wait()

        iv_cb = []
        for ch in range(d // 16):
            row = []
            for blo in range(8):
                cc = lane + ch * 16
                row.append((cc >> 3) * (_PB * 8 * _LANES)
                           + (cc & 7) * _LANES + blo)
            iv_cb.append(row)
        window = (d // 8 - 1) * (_PB * 8 * _LANES) + 7 * _LANES + 8 + 16

        def transpose(b):
            def lbody(bl8, carry):
                base = bl8 * 8
                for j2 in range(_PB):
                    off = j2 * (8 * _LANES) + base
                    dst = obufs[b].at[pl.ds(off, window)]
                    for ch in range(d // 16):
                        vecs = [row_bufs[b][j2 * _LANES + base + blo,
                                            pl.ds(ch * 16, 16)]
                                for blo in range(8)]
                        for blo in range(8):
                            plsc.store_scatter(dst, [iv_cb[ch][blo]],
                                               vecs[blo])
                return carry
            plsc.parallel_loop(0, _LANES // 8, carry=jnp.int32(0))(lbody)

        idx_start(0, 0)
        idx_start(1, 1)
        idx_wait(0)
        gather_start(0)

        def outer_body(o, carry):
            for b in range(2):
                s = o * 2 + b
                nb = 1 - b

                @pl.when(s + 1 < n_stages)
                def _():
                    idx_wait(nb)
                    gather_start(nb)

                gather_wait(b)

                @pl.when(s >= 2)
                def _():
                    out_wait(b)

                transpose(b)
                out_start(s, b)

                @pl.when(s + 2 < n_stages)
                def _():
                    idx_start(s + 2, b)
            return carry

        lax.fori_loop(0, n_stages // 2, outer_body, 0)
        out_wait(0)
        out_wait(1)

    return k(idx_flat, table)


def kernel(tokens, embeddings):
    bsz, hist = tokens.shape
    v, d = embeddings.shape
    n_pairs = hist * (bsz // _LANES)
    idx_flat = jnp.transpose(tokens).reshape(bsz * hist).astype(jnp.int32)
    out_lin = _gather_sc(idx_flat, embeddings, n_pairs, d)
    out = out_lin.reshape(hist, _CG, bsz // _LANES, 8, _LANES)
    out = out.transpose(2, 4, 0, 1, 3).reshape(bsz, hist, d)
    return out


# speedup vs baseline: 1.0477x; 1.0004x over previous
import functools

import jax
import jax.numpy as jnp
from jax import lax
from jax.experimental import pallas as pl
from jax.experimental.pallas import tpu as pltpu
from jax.experimental.pallas import tpu_sc as plsc

_INFO = plsc.get_sparse_core_info()
_NC = _INFO.num_cores
_NS = _INFO.num_subcores
_NW = _NC * _NS

_PB = 4
_LANES = 128
_CG = 4


def _gather_sc(idx_flat, table, n_pairs, d):
    pairs_per_w = n_pairs // _NW
    n_stages = pairs_per_w // _PB
    rows_per_stage = _PB * _LANES
    obuf_len = _CG * _PB * 8 * _LANES
    mesh = plsc.VectorSubcoreMesh(core_axis_name="c", subcore_axis_name="s")

    @functools.partial(
        pl.kernel,
        mesh=mesh,
        out_type=jax.ShapeDtypeStruct((n_pairs // _LANES, _CG, _LANES * 8 * _LANES),
                                      jnp.float32),
        scratch_types=(
            [pltpu.VMEM((rows_per_stage,), jnp.int32) for _ in range(2)]
            + [pltpu.VMEM((rows_per_stage, d), jnp.float32) for _ in range(2)]
            + [pltpu.VMEM((obuf_len + _LANES + 16,), jnp.float32)
               for _ in range(2)]
            + [pltpu.SemaphoreType.DMA for _ in range(6)]
        ),
        compiler_params=pltpu.CompilerParams(use_tc_tiling_on_sc=False,
                                             needs_layout_passes=False),
    )
    def k(idx_hbm, table_hbm, out_hbm, *scr):
        idx_bufs, row_bufs, obufs = scr[0:2], scr[2:4], scr[4:6]
        isems, gsems, osems = scr[6:8], scr[8:10], scr[10:12]

        wid = lax.axis_index("s") * _NC + lax.axis_index("c")
        p0 = wid * pairs_per_w
        lane = lax.iota(jnp.int32, 16)

        def idx_start(s, b):
            pltpu.async_copy(
                idx_hbm.at[pl.ds((p0 + s * _PB) * _LANES, rows_per_stage)],
                idx_bufs[b], isems[b])

        def idx_wait(b):
            pltpu.make_async_copy(
                idx_hbm.at[pl.ds(0, rows_per_stage)], idx_bufs[b],
                isems[b]).wait()

        def gather_start(b):
            pltpu.async_copy(table_hbm.at[idx_bufs[b]], row_bufs[b], gsems[b])

        def gather_wait(b):
            pltpu.make_async_copy(
                table_hbm.at[idx_bufs[b]], row_bufs[b], gsems[b]).wait()

        def out_start(s, b):
            p = p0 + s * _PB
            h = p // _LANES
            btile = p % _LANES
            for c4 in range(_CG):
                pltpu.async_copy(
                    obufs[b].at[pl.ds(c4 * _PB * 8 * _LANES, _PB * 8 * _LANES)],
                    out_hbm.at[h, c4, pl.ds(btile * 8 * _LANES,
                                            _PB * 8 * _LANES)],
                    osems[b])

        def out_wait(b):
            for c4 in range(_CG):
                pltpu.make_async_copy(
                    obufs[b].at[pl.ds(0, _PB * 8 * _LANES)],
                    out_hbm.at[0, 0, pl.ds(0, _PB * 8 * _LANES)],
                    osems[b]).wait()

        iv_cb = []
        for ch in range(d // 16):
            row = []
            for blo in range(8):
                cc = lane + ch * 16
                row.append((cc >> 3) * (_PB * 8 * _LANES)
                           + (cc & 7) * _LANES + blo)
            iv_cb.append(row)
        window = (d // 8 - 1) * (_PB * 8 * _LANES) + 7 * _LANES + 8 + 16

        def transpose(b):
            def lbody(bl8, carry):
                base = bl8 * 8
                for j2 in range(_PB):
                    off = j2 * (8 * _LANES) + base
                    dst = obufs[b].at[pl.ds(off, window)]
                    for ch in range(d // 16):
                        vecs = [row_bufs[b][j2 * _LANES + base + blo,
                                            pl.ds(ch * 16, 16)]
                                for blo in range(8)]
                        for blo in range(8):
                            plsc.store_scatter(dst, [iv_cb[ch][blo]],
                                               vecs[blo])
                return carry
            plsc.parallel_loop(0, _LANES // 8, carry=jnp.int32(0))(lbody)

        idx_start(0, 0)
        idx_start(1, 1)
        idx_wait(0)
        gather_start(0)

        def outer_body(o, carry):
            for b in range(2):
                s = o * 2 + b
                nb = 1 - b

                @pl.when(s + 1 < n_stages)
                def _():
                    idx_wait(nb)
                    gather_start(nb)

                gather_wait(b)

                @pl.when(s >= 2)
                def _():
                    out_wait(b)

                transpose(b)
                out_start(s, b)

                @pl.when(s + 2 < n_stages)
                def _():
                    idx_start(s + 2, b)
            return carry

        lax.fori_loop(0, n_stages // 2, outer_body, 0)
        out_wait(0)
        out_wait(1)

    return k(idx_flat, table)


def kernel(tokens, embeddings):
    bsz, hist = tokens.shape
    v, d = embeddings.shape
    n_pairs = hist * (bsz // _LANES)
    idx_flat = jnp.transpose(tokens).reshape(bsz * hist).astype(jnp.int32)
    out_lin = _gather_sc(idx_flat, embeddings, n_pairs, d)
    out = out_lin.reshape(hist, _CG, bsz // _LANES, 8, _LANES)
    out = out.transpose(2, 4, 0, 1, 3).reshape(bsz, hist, d)
    return out
